# Initial kernel scaffold; baseline (speedup 1.0000x reference)
import jax
import jax.numpy as jnp
from jax.experimental import pallas as pl


def kernel(x, W_qkv, W_o, W_qi, W_ki, W_wi):
    raise NotImplementedError("write your pallas kernel here")


# speedup vs baseline: 9.5033x; 9.5033x over previous
import functools

import jax
import jax.numpy as jnp
from jax.experimental import pallas as pl

D_MODEL = 1024
N_HEADS = 16
D_K = D_MODEL // N_HEADS
H_I = 4
D_I = 64
TOP_K = 512
BASE = 10000.0
NEG = -1e9

_MT = 256
_NT = 256


def _proj_kernel(x_ref, w_ref, cc_ref, ss_ref, o_ref):
    j = pl.program_id(1)
    acc = jnp.dot(x_ref[...], w_ref[...], preferred_element_type=jnp.float32)
    lane = jax.lax.broadcasted_iota(jnp.int32, acc.shape, 1)
    even = (lane % 2) == 0
    xswap = jnp.where(even, jnp.roll(acc, -1, axis=1), jnp.roll(acc, 1, axis=1))
    roped = acc * cc_ref[...] + xswap * ss_ref[...]
    n_rope_tiles = (2 * D_MODEL) // _NT
    o_ref[...] = jnp.where(j < n_rope_tiles, roped, acc)


def _mask_kernel(nq, qi_ref, kit_ref, wi_ref, o_ref):
    qq = pl.program_id(1)
    s = kit_ref.shape[-1]
    score = None
    for h in range(H_I):
        dh = jnp.dot(qi_ref[:, h * D_I:(h + 1) * D_I], kit_ref[0],
                     preferred_element_type=jnp.float32)
        term = jnp.maximum(dh, 0.0) * wi_ref[:, h:h + 1]
        score = term if score is None else score + term
    row = qq * _MT + jax.lax.broadcasted_iota(jnp.int32, (_MT, s), 0)
    col = jax.lax.broadcasted_iota(jnp.int32, (_MT, s), 1)
    score = jnp.where(col > row, NEG, score)

    int_min = jnp.int32(-(2**31))
    bits = jax.lax.bitcast_convert_type(score, jnp.int32)
    keys = jnp.where(bits < 0, bits ^ jnp.int32(0x7FFFFFFF), bits)

    def vbody(jj, tu):
        trial = tu | jnp.left_shift(jnp.int32(1), 31 - jj)
        cnt = jnp.sum((keys >= (trial ^ _INT_MIN)).astype(jnp.int32),
                      axis=1, keepdims=True)
        return jnp.where(cnt >= TOP_K, trial, tu)

    tu = jax.lax.fori_loop(0, 32, vbody, jnp.zeros((_MT, 1), jnp.int32))
    tkey = tu ^ _INT_MIN
    gt = keys > tkey
    eqm = keys == tkey
    r = TOP_K - jnp.sum(gt.astype(jnp.int32), axis=1, keepdims=True)

    def ibody(jj, c):
        trial = c | jnp.left_shift(jnp.int32(1), 11 - jj)
        cnt = jnp.sum((eqm & (col < trial)).astype(jnp.int32),
                      axis=1, keepdims=True)
        return jnp.where(cnt < r, trial, c)

    c = jax.lax.fori_loop(0, 12, ibody, jnp.zeros((_MT, 1), jnp.int32))
    sel = gt | (eqm & (col < c + 1))
    o_ref[...] = sel.astype(jnp.int8)


def _attn_kernel(q_ref, kt_ref, v_ref, m_ref, o_ref):
    logits = jnp.dot(q_ref[0, 0], kt_ref[0, 0],
                     preferred_element_type=jnp.float32) * 0.125
    logits = logits + (m_ref[0].astype(jnp.float32) - 1.0) * 1e9
    mx = jnp.max(logits, axis=1, keepdims=True)
    e = jnp.exp(logits - mx)
    p = e / jnp.sum(e, axis=1, keepdims=True)
    o_ref[0, 0] = jnp.dot(p, v_ref[0, 0], preferred_element_type=jnp.float32)


def _matmul_kernel(x_ref, w_ref, o_ref):
    o_ref[...] = jnp.dot(x_ref[...], w_ref[...],
                         preferred_element_type=jnp.float32)


def kernel(x, W_qkv, W_o, W_qi, W_ki, W_wi):
    b, s, _ = x.shape
    bs = b * s
    nq = s // _MT
    nm = bs // _MT

    x_flat = x.reshape(bs, D_MODEL)

    n_real = 3 * D_MODEL + H_I * D_I + D_I + H_I
    n_pad = ((n_real + _NT - 1) // _NT) * _NT
    W_cat = jnp.concatenate(
        [W_qkv, W_qi, W_ki, W_wi,
         jnp.zeros((n_pad - n_real, D_MODEL), jnp.float32)], axis=0)
    nn = n_pad // _NT

    theta = 1.0 / (BASE ** (jnp.arange(0, D_K, 2, dtype=jnp.float32) / D_K))
    th_lane = jnp.tile(jnp.repeat(theta, 2), _NT // D_K)
    t = (jnp.arange(bs, dtype=jnp.float32) % s)
    ang = t[:, None] * th_lane[None, :]
    lane = jnp.arange(_NT)
    cc = jnp.cos(ang)
    ss = jnp.sin(ang) * jnp.where(lane % 2 == 1, 1.0, -1.0)[None, :]

    proj = pl.pallas_call(
        _proj_kernel,
        grid=(nm, nn),
        in_specs=[
            pl.BlockSpec((_MT, D_MODEL), lambda i, j: (i, 0)),
            pl.BlockSpec((D_MODEL, _NT), lambda i, j: (0, j)),
            pl.BlockSpec((_MT, _NT), lambda i, j: (i, 0)),
            pl.BlockSpec((_MT, _NT), lambda i, j: (i, 0)),
        ],
        out_specs=pl.BlockSpec((_MT, _NT), lambda i, j: (i, j)),
        out_shape=jax.ShapeDtypeStruct((bs, n_pad), jnp.float32),
    )(x_flat, W_cat.T, cc, ss)

    qi = proj[:, 3 * D_MODEL:3 * D_MODEL + H_I * D_I]
    ki = proj[:, 3 * D_MODEL + H_I * D_I:3 * D_MODEL + H_I * D_I + D_I]
    wi_pad = proj[:, 3 * D_MODEL + H_I * D_I + D_I:
                  3 * D_MODEL + H_I * D_I + D_I + 128]
    kit = jnp.transpose(ki.reshape(b, s, D_I), (0, 2, 1))

    mask = pl.pallas_call(
        functools.partial(_mask_kernel, nq),
        grid=(b, nq),
        in_specs=[
            pl.BlockSpec((_MT, H_I * D_I), lambda bb, qq: (bb * nq + qq, 0)),
            pl.BlockSpec((1, D_I, s), lambda bb, qq: (bb, 0, 0)),
            pl.BlockSpec((_MT, 128), lambda bb, qq: (bb * nq + qq, 0)),
        ],
        out_specs=pl.BlockSpec((_MT, s), lambda bb, qq: (bb * nq + qq, 0)),
        out_shape=jax.ShapeDtypeStruct((bs, s), jnp.int8),
    )(qi, kit, wi_pad)
    mask3 = mask.reshape(b, s, s)

    qkv3 = proj[:, :3 * D_MODEL].reshape(b, s, 3, N_HEADS, D_K)
    Q = jnp.transpose(qkv3[:, :, 0], (0, 2, 1, 3))
    KT = jnp.transpose(qkv3[:, :, 1], (0, 2, 3, 1))
    V = jnp.transpose(qkv3[:, :, 2], (0, 2, 1, 3))

    attn = pl.pallas_call(
        _attn_kernel,
        grid=(b, N_HEADS, nq),
        in_specs=[
            pl.BlockSpec((1, 1, _MT, D_K), lambda bb, hh, qq: (bb, hh, qq, 0)),
            pl.BlockSpec((1, 1, D_K, s), lambda bb, hh, qq: (bb, hh, 0, 0)),
            pl.BlockSpec((1, 1, s, D_K), lambda bb, hh, qq: (bb, hh, 0, 0)),
            pl.BlockSpec((1, _MT, s), lambda bb, hh, qq: (bb, qq, 0)),
        ],
        out_specs=pl.BlockSpec((1, 1, _MT, D_K),
                               lambda bb, hh, qq: (bb, hh, qq, 0)),
        out_shape=jax.ShapeDtypeStruct((b, N_HEADS, s, D_K), jnp.float32),
    )(Q, KT, V, mask3)

    attn_flat = jnp.transpose(attn, (0, 2, 1, 3)).reshape(bs, D_MODEL)

    out = pl.pallas_call(
        _matmul_kernel,
        grid=(nm, D_MODEL // _NT),
        in_specs=[
            pl.BlockSpec((_MT, D_MODEL), lambda i, j: (i, 0)),
            pl.BlockSpec((D_MODEL, _NT), lambda i, j: (0, j)),
        ],
        out_specs=pl.BlockSpec((_MT, _NT), lambda i, j: (i, j)),
        out_shape=jax.ShapeDtypeStruct((bs, D_MODEL), jnp.float32),
    )(attn_flat, W_o.T)

    return out.reshape(b, s, D_MODEL)


# speedup vs baseline: 16.7365x; 1.7611x over previous
import jax
import jax.numpy as jnp
from jax.experimental import pallas as pl

D_MODEL = 1024
N_HEADS = 16
D_K = D_MODEL // N_HEADS
H_I = 4
D_I = 64
TOP_K = 512
BASE = 10000.0
NEG = -1e9

_MT = 256
_PM = 1024
_PN = 512

_TRT = (((1,), (1,)), ((), ()))


def _proj_kernel(x_ref, w_ref, cc_ref, ss_ref, o_ref):
    j = pl.program_id(1)
    acc = jax.lax.dot_general(x_ref[...], w_ref[...], _TRT,
                              preferred_element_type=jnp.float32)
    lane = jax.lax.broadcasted_iota(jnp.int32, acc.shape, 1)
    even = (lane % 2) == 0
    xswap = jnp.where(even, jnp.roll(acc, -1, axis=1), jnp.roll(acc, 1, axis=1))
    roped = acc * cc_ref[...] + xswap * ss_ref[...]
    n_rope_tiles = (2 * D_MODEL) // _PN
    o_ref[...] = jnp.where(j < n_rope_tiles, roped, acc)


def _mask_kernel(qi_ref, ki_ref, wi_ref, o_ref):
    qq = pl.program_id(1)
    s = ki_ref.shape[0]
    score = None
    for h in range(H_I):
        dh = jax.lax.dot_general(qi_ref[:, h * D_I:(h + 1) * D_I],
                                 ki_ref[:, :D_I],
                                 _TRT, preferred_element_type=jnp.float32)
        term = jnp.maximum(dh, 0.0) * wi_ref[:, D_I + h:D_I + h + 1]
        score = term if score is None else score + term
    row = qq * _MT + jax.lax.broadcasted_iota(jnp.int32, (_MT, s), 0)
    col = jax.lax.broadcasted_iota(jnp.int32, (_MT, s), 1)
    score = jnp.where(col > row, NEG, score)

    int_min = jnp.int32(-(2**31))
    bits = jax.lax.bitcast_convert_type(score, jnp.int32)
    keys = jnp.where(bits < 0, bits ^ jnp.int32(0x7FFFFFFF), bits)

    def vbody(jj, tu):
        trial = tu | jnp.left_shift(jnp.int32(1), 31 - jj)
        cnt = jnp.sum((keys >= (trial ^ int_min)).astype(jnp.int32),
                      axis=1, keepdims=True)
        return jnp.where(cnt >= TOP_K, trial, tu)

    tu = jax.lax.fori_loop(0, 32, vbody, jnp.zeros((_MT, 1), jnp.int32))
    tkey = tu ^ int_min
    gt = keys > tkey
    eqm = keys == tkey
    r = TOP_K - jnp.sum(gt.astype(jnp.int32), axis=1, keepdims=True)

    def ibody(jj, c):
        trial = c | jnp.left_shift(jnp.int32(1), 11 - jj)
        cnt = jnp.sum((eqm & (col < trial)).astype(jnp.int32),
                      axis=1, keepdims=True)
        return jnp.where(cnt < r, trial, c)

    c = jax.lax.fori_loop(0, 12, ibody, jnp.zeros((_MT, 1), jnp.int32))
    sel = gt | (eqm & (col < c + 1))
    o_ref[...] = sel.astype(jnp.int8)


def _attn_kernel(q_ref, k_ref, v_ref, m_ref, o_ref):
    madd = (m_ref[...].astype(jnp.float32) - 1.0) * 1e9
    for h in range(N_HEADS):
        sl = slice(h * D_K, (h + 1) * D_K)
        logits = jax.lax.dot_general(q_ref[:, sl], k_ref[:, sl], _TRT,
                                     preferred_element_type=jnp.float32)
        logits = logits * 0.125 + madd
        mx = jnp.max(logits, axis=1, keepdims=True)
        e = jnp.exp(logits - mx)
        p = e / jnp.sum(e, axis=1, keepdims=True)
        o_ref[:, sl] = jnp.dot(p, v_ref[:, sl],
                               preferred_element_type=jnp.float32)


def _matmul_kernel(x_ref, w_ref, o_ref):
    o_ref[...] = jax.lax.dot_general(x_ref[...], w_ref[...], _TRT,
                                     preferred_element_type=jnp.float32)


def kernel(x, W_qkv, W_o, W_qi, W_ki, W_wi):
    b, s, _ = x.shape
    bs = b * s
    nq = s // _MT
    nm = bs // _PM

    x_flat = x.reshape(bs, D_MODEL)

    n_real = 3 * D_MODEL + H_I * D_I + D_I + H_I
    n_pad = ((n_real + _PN - 1) // _PN) * _PN
    W_cat = jnp.concatenate(
        [W_qkv, W_qi, W_ki, W_wi,
         jnp.zeros((n_pad - n_real, D_MODEL), jnp.float32)], axis=0)
    nn = n_pad // _PN

    theta = 1.0 / (BASE ** (jnp.arange(0, D_K, 2, dtype=jnp.float32) / D_K))
    th_lane = jnp.tile(jnp.repeat(theta, 2), _PN // D_K)
    t = (jnp.arange(bs, dtype=jnp.float32) % s)
    ang = t[:, None] * th_lane[None, :]
    lane = jnp.arange(_PN)
    cc = jnp.cos(ang)
    ss = jnp.sin(ang) * jnp.where(lane % 2 == 1, 1.0, -1.0)[None, :]

    proj = pl.pallas_call(
        _proj_kernel,
        grid=(nm, nn),
        in_specs=[
            pl.BlockSpec((_PM, D_MODEL), lambda i, j: (i, 0)),
            pl.BlockSpec((_PN, D_MODEL), lambda i, j: (j, 0)),
            pl.BlockSpec((_PM, _PN), lambda i, j: (i, 0)),
            pl.BlockSpec((_PM, _PN), lambda i, j: (i, 0)),
        ],
        out_specs=pl.BlockSpec((_PM, _PN), lambda i, j: (i, j)),
        out_shape=jax.ShapeDtypeStruct((bs, n_pad), jnp.float32),
    )(x_flat, W_cat, cc, ss)

    qi_cb = (3 * D_MODEL) // 256
    kw_cb = (3 * D_MODEL + H_I * D_I) // 128

    mask = pl.pallas_call(
        _mask_kernel,
        grid=(b, nq),
        in_specs=[
            pl.BlockSpec((_MT, H_I * D_I), lambda bb, qq: (bb * nq + qq, qi_cb)),
            pl.BlockSpec((s, 128), lambda bb, qq: (bb, kw_cb)),
            pl.BlockSpec((_MT, 128), lambda bb, qq: (bb * nq + qq, kw_cb)),
        ],
        out_specs=pl.BlockSpec((_MT, s), lambda bb, qq: (bb * nq + qq, 0)),
        out_shape=jax.ShapeDtypeStruct((bs, s), jnp.int8),
    )(proj, proj, proj)

    attn = pl.pallas_call(
        _attn_kernel,
        grid=(b, nq),
        in_specs=[
            pl.BlockSpec((_MT, D_MODEL), lambda bb, qq: (bb * nq + qq, 0)),
            pl.BlockSpec((s, D_MODEL), lambda bb, qq: (bb, 1)),
            pl.BlockSpec((s, D_MODEL), lambda bb, qq: (bb, 2)),
            pl.BlockSpec((_MT, s), lambda bb, qq: (bb * nq + qq, 0)),
        ],
        out_specs=pl.BlockSpec((_MT, D_MODEL), lambda bb, qq: (bb * nq + qq, 0)),
        out_shape=jax.ShapeDtypeStruct((bs, D_MODEL), jnp.float32),
    )(proj, proj, proj, mask)

    out = pl.pallas_call(
        _matmul_kernel,
        grid=(nm, D_MODEL // _PN),
        in_specs=[
            pl.BlockSpec((_PM, D_MODEL), lambda i, j: (i, 0)),
            pl.BlockSpec((_PN, D_MODEL), lambda i, j: (j, 0)),
        ],
        out_specs=pl.BlockSpec((_PM, _PN), lambda i, j: (i, j)),
        out_shape=jax.ShapeDtypeStruct((bs, D_MODEL), jnp.float32),
    )(attn, W_o)

    return out.reshape(b, s, D_MODEL)


# speedup vs baseline: 21.2490x; 1.2696x over previous
import functools

import jax
import jax.numpy as jnp
from jax.experimental import pallas as pl

D_MODEL = 1024
N_HEADS = 16
D_K = D_MODEL // N_HEADS
H_I = 4
D_I = 64
TOP_K = 512
BASE = 10000.0
NEG = -1e9
KEY_NEG = -1315859241

_MT = 256
_PM = 1024
_PN = 512

_TRT = (((1,), (1,)), ((), ()))


def _proj_kernel(x_ref, w_ref, cc_ref, ss_ref, o_ref):
    j = pl.program_id(1)
    acc = jax.lax.dot_general(x_ref[...], w_ref[...], _TRT,
                              preferred_element_type=jnp.float32)
    lane = jax.lax.broadcasted_iota(jnp.int32, acc.shape, 1)
    even = (lane % 2) == 0
    xswap = jnp.where(even, jnp.roll(acc, -1, axis=1), jnp.roll(acc, 1, axis=1))
    roped = acc * cc_ref[...] + xswap * ss_ref[...]
    n_rope_tiles = (2 * D_MODEL) // _PN
    o_ref[...] = jnp.where(j < n_rope_tiles, roped, acc)


def _mask_kernel(q0, w, s, qi_ref, ki_ref, wi_ref, o_ref):
    jj = pl.program_id(1)
    score = None
    for h in range(H_I):
        dh = jax.lax.dot_general(qi_ref[0, :, h * D_I:(h + 1) * D_I],
                                 ki_ref[0, :, :D_I],
                                 _TRT, preferred_element_type=jnp.float32)
        term = jnp.maximum(dh, 0.0) * wi_ref[0, :, D_I + h:D_I + h + 1]
        score = term if score is None else score + term
    row = (q0 + jj) * _MT + jax.lax.broadcasted_iota(jnp.int32, (_MT, w), 0)
    col = jax.lax.broadcasted_iota(jnp.int32, (_MT, w), 1)
    score = jnp.where(col > row, NEG, score)

    int_min = jnp.int32(-(2**31))
    key_neg = jnp.int32(KEY_NEG)
    tail = jnp.int32(s - w)
    bits = jax.lax.bitcast_convert_type(score, jnp.int32)
    keys = jnp.where(bits < 0, bits ^ jnp.int32(0x7FFFFFFF), bits)

    def vbody(i, tu):
        trial = tu | jnp.left_shift(jnp.int32(1), 31 - i)
        cmp = trial ^ int_min
        cnt = jnp.sum((keys >= cmp).astype(jnp.int32), axis=1, keepdims=True)
        cnt = cnt + tail * (key_neg >= cmp).astype(jnp.int32)
        return jnp.where(cnt >= TOP_K, trial, tu)

    tu = jax.lax.fori_loop(0, 32, vbody, jnp.zeros((_MT, 1), jnp.int32))
    tkey = tu ^ int_min
    gt = keys > tkey
    eqm = keys == tkey
    c_gt = jnp.sum(gt.astype(jnp.int32), axis=1, keepdims=True)
    c_gt = c_gt + tail * (key_neg > tkey).astype(jnp.int32)
    r = TOP_K - c_gt

    nbits = (w - 1).bit_length()

    def ibody(i, c):
        trial = c | jnp.left_shift(jnp.int32(1), nbits - 1 - i)
        cnt = jnp.sum((eqm & (col < trial)).astype(jnp.int32),
                      axis=1, keepdims=True)
        return jnp.where(cnt < r, trial, c)

    c = jax.lax.fori_loop(0, nbits, ibody, jnp.zeros((_MT, 1), jnp.int32))
    sel = gt | (eqm & (col < c + 1))
    o_ref[0, :, :] = sel.astype(jnp.int8)


def _attn_kernel(q_ref, k_ref, v_ref, m_ref, o_ref):
    madd = (m_ref[0].astype(jnp.float32) - 1.0) * 1e9
    for h in range(N_HEADS):
        sl = slice(h * D_K, (h + 1) * D_K)
        logits = jax.lax.dot_general(q_ref[0, :, sl] * 0.125, k_ref[0, :, sl],
                                     _TRT, preferred_element_type=jnp.float32)
        logits = logits + madd
        mx = jnp.max(logits, axis=1, keepdims=True)
        e = jnp.exp(logits - mx)
        o = jnp.dot(e, v_ref[0, :, sl], preferred_element_type=jnp.float32)
        o_ref[0, :, sl] = o / jnp.sum(e, axis=1, keepdims=True)


def _matmul_kernel(x_ref, w_ref, o_ref):
    o_ref[...] = jax.lax.dot_general(x_ref[...], w_ref[...], _TRT,
                                     preferred_element_type=jnp.float32)


def kernel(x, W_qkv, W_o, W_qi, W_ki, W_wi):
    b, s, _ = x.shape
    bs = b * s
    nq = s // _MT
    nm = bs // _PM

    x_flat = x.reshape(bs, D_MODEL)

    n_real = 3 * D_MODEL + H_I * D_I + D_I + H_I
    n_pad = ((n_real + _PN - 1) // _PN) * _PN
    W_cat = jnp.concatenate(
        [W_qkv, W_qi, W_ki, W_wi,
         jnp.zeros((n_pad - n_real, D_MODEL), jnp.float32)], axis=0)
    nn = n_pad // _PN

    theta = 1.0 / (BASE ** (jnp.arange(0, D_K, 2, dtype=jnp.float32) / D_K))
    th_lane = jnp.tile(jnp.repeat(theta, 2), _PN // D_K)
    t = (jnp.arange(bs, dtype=jnp.float32) % s)
    ang = t[:, None] * th_lane[None, :]
    lane = jnp.arange(_PN)
    cc = jnp.cos(ang)
    ss = jnp.sin(ang) * jnp.where(lane % 2 == 1, 1.0, -1.0)[None, :]

    proj = pl.pallas_call(
        _proj_kernel,
        grid=(nm, nn),
        in_specs=[
            pl.BlockSpec((_PM, D_MODEL), lambda i, j: (i, 0)),
            pl.BlockSpec((_PN, D_MODEL), lambda i, j: (j, 0)),
            pl.BlockSpec((_PM, _PN), lambda i, j: (i, 0)),
            pl.BlockSpec((_PM, _PN), lambda i, j: (i, 0)),
        ],
        out_specs=pl.BlockSpec((_PM, _PN), lambda i, j: (i, j)),
        out_shape=jax.ShapeDtypeStruct((bs, n_pad), jnp.float32),
    )(x_flat, W_cat, cc, ss)

    proj3 = proj.reshape(b, s, n_pad)

    qi_cb = (3 * D_MODEL) // 256
    kw_cb = (3 * D_MODEL + H_I * D_I) // 128

    groups = []
    for qq in range(nq):
        w = min(s, max(TOP_K, (qq + 1) * _MT))
        if groups and groups[-1][2] == w:
            q0, gn, _ = groups[-1]
            groups[-1] = (q0, gn + 1, w)
        else:
            groups.append((qq, 1, w))

    attn_parts = []
    for q0, gn, w in groups:
        mask_g = pl.pallas_call(
            functools.partial(_mask_kernel, q0, w, s),
            grid=(b, gn),
            in_specs=[
                pl.BlockSpec((1, _MT, H_I * D_I),
                             lambda bb, j, q0=q0: (bb, q0 + j, qi_cb)),
                pl.BlockSpec((1, w, 128), lambda bb, j: (bb, 0, kw_cb)),
                pl.BlockSpec((1, _MT, 128),
                             lambda bb, j, q0=q0: (bb, q0 + j, kw_cb)),
            ],
            out_specs=pl.BlockSpec((1, _MT, w), lambda bb, j: (bb, j, 0)),
            out_shape=jax.ShapeDtypeStruct((b, gn * _MT, w), jnp.int8),
        )(proj3, proj3, proj3)

        attn_g = pl.pallas_call(
            _attn_kernel,
            grid=(b, gn),
            in_specs=[
                pl.BlockSpec((1, _MT, D_MODEL),
                             lambda bb, j, q0=q0: (bb, q0 + j, 0)),
                pl.BlockSpec((1, w, D_MODEL), lambda bb, j: (bb, 0, 1)),
                pl.BlockSpec((1, w, D_MODEL), lambda bb, j: (bb, 0, 2)),
                pl.BlockSpec((1, _MT, w), lambda bb, j: (bb, j, 0)),
            ],
            out_specs=pl.BlockSpec((1, _MT, D_MODEL),
                                   lambda bb, j: (bb, j, 0)),
            out_shape=jax.ShapeDtypeStruct((b, gn * _MT, D_MODEL), jnp.float32),
        )(proj3, proj3, proj3, mask_g)
        attn_parts.append(attn_g)

    attn = jnp.concatenate(attn_parts, axis=1).reshape(bs, D_MODEL)

    out = pl.pallas_call(
        _matmul_kernel,
        grid=(nm, D_MODEL // _PN),
        in_specs=[
            pl.BlockSpec((_PM, D_MODEL), lambda i, j: (i, 0)),
            pl.BlockSpec((_PN, D_MODEL), lambda i, j: (j, 0)),
        ],
        out_specs=pl.BlockSpec((_PM, _PN), lambda i, j: (i, j)),
        out_shape=jax.ShapeDtypeStruct((bs, D_MODEL), jnp.float32),
    )(attn, W_o)

    return out.reshape(b, s, D_MODEL)


# speedup vs baseline: 21.2950x; 1.0022x over previous
import functools

import jax
import jax.numpy as jnp
from jax.experimental import pallas as pl

D_MODEL = 1024
N_HEADS = 16
D_K = D_MODEL // N_HEADS
H_I = 4
D_I = 64
TOP_K = 512
BASE = 10000.0
NEG = -1e9
KEY_NEG = -1315859241

_MT = 256
_PM = 1024
_PN = 512

_TRT = (((1,), (1,)), ((), ()))


def _proj_kernel(x_ref, w_ref, cc_ref, ss_ref, o_ref):
    j = pl.program_id(1)
    acc = jax.lax.dot_general(x_ref[...], w_ref[...], _TRT,
                              preferred_element_type=jnp.float32)
    lane = jax.lax.broadcasted_iota(jnp.int32, acc.shape, 1)
    even = (lane % 2) == 0
    xswap = jnp.where(even, jnp.roll(acc, -1, axis=1), jnp.roll(acc, 1, axis=1))
    roped = acc * cc_ref[...] + xswap * ss_ref[...]
    n_rope_tiles = (2 * D_MODEL) // _PN
    o_ref[...] = jnp.where(j < n_rope_tiles, roped, acc)


def _fused_kernel(q0, w, s, q_ref, qi_ref, qw_ref, k_ref, v_ref, kw_ref, o_ref):
    jj = pl.program_id(1)
    score = None
    for h in range(H_I):
        dh = jax.lax.dot_general(qi_ref[0, :, h * D_I:(h + 1) * D_I],
                                 kw_ref[0, :, :D_I],
                                 _TRT, preferred_element_type=jnp.float32)
        term = jnp.maximum(dh, 0.0) * qw_ref[0, :, D_I + h:D_I + h + 1]
        score = term if score is None else score + term
    row = (q0 + jj) * _MT + jax.lax.broadcasted_iota(jnp.int32, (_MT, w), 0)
    col = jax.lax.broadcasted_iota(jnp.int32, (_MT, w), 1)
    score = jnp.where(col > row, NEG, score)

    int_min = jnp.int32(-(2**31))
    key_neg = jnp.int32(KEY_NEG)
    tail = jnp.int32(s - w)
    bits = jax.lax.bitcast_convert_type(score, jnp.int32)
    keys = jnp.where(bits < 0, bits ^ jnp.int32(0x7FFFFFFF), bits)

    def vbody(i, tu):
        trial = tu | jnp.left_shift(jnp.int32(1), 31 - i)
        cmp = trial ^ int_min
        cnt = jnp.sum((keys >= cmp).astype(jnp.int32), axis=1, keepdims=True)
        cnt = cnt + tail * (key_neg >= cmp).astype(jnp.int32)
        return jnp.where(cnt >= TOP_K, trial, tu)

    tu = jax.lax.fori_loop(0, 32, vbody, jnp.zeros((_MT, 1), jnp.int32))
    tkey = tu ^ int_min
    gt = keys > tkey
    eqm = keys == tkey
    c_gt = jnp.sum(gt.astype(jnp.int32), axis=1, keepdims=True)
    c_gt = c_gt + tail * (key_neg > tkey).astype(jnp.int32)
    r = TOP_K - c_gt

    nbits = (w - 1).bit_length()

    def ibody(i, c):
        trial = c | jnp.left_shift(jnp.int32(1), nbits - 1 - i)
        cnt = jnp.sum((eqm & (col < trial)).astype(jnp.int32),
                      axis=1, keepdims=True)
        return jnp.where(cnt < r, trial, c)

    c = jax.lax.fori_loop(0, nbits, ibody, jnp.zeros((_MT, 1), jnp.int32))
    sel = gt | (eqm & (col < c + 1))
    madd = jnp.where(sel, 0.0, NEG)

    for h in range(N_HEADS):
        sl = slice(h * D_K, (h + 1) * D_K)
        logits = jax.lax.dot_general(q_ref[0, :, sl] * 0.125, k_ref[0, :, sl],
                                     _TRT, preferred_element_type=jnp.float32)
        logits = logits + madd
        mx = jnp.max(logits, axis=1, keepdims=True)
        e = jnp.exp(logits - mx)
        o = jnp.dot(e, v_ref[0, :, sl], preferred_element_type=jnp.float32)
        o_ref[0, :, sl] = o / jnp.sum(e, axis=1, keepdims=True)


def _matmul_kernel(x_ref, w_ref, o_ref):
    o_ref[...] = jax.lax.dot_general(x_ref[...], w_ref[...], _TRT,
                                     preferred_element_type=jnp.float32)


def kernel(x, W_qkv, W_o, W_qi, W_ki, W_wi):
    b, s, _ = x.shape
    bs = b * s
    nq = s // _MT
    nm = bs // _PM

    x_flat = x.reshape(bs, D_MODEL)

    n_real = 3 * D_MODEL + H_I * D_I + D_I + H_I
    n_pad = ((n_real + _PN - 1) // _PN) * _PN
    W_cat = jnp.concatenate(
        [W_qkv, W_qi, W_ki, W_wi,
         jnp.zeros((n_pad - n_real, D_MODEL), jnp.float32)], axis=0)
    nn = n_pad // _PN

    theta = 1.0 / (BASE ** (jnp.arange(0, D_K, 2, dtype=jnp.float32) / D_K))
    th_lane = jnp.tile(jnp.repeat(theta, 2), _PN // D_K)
    t = (jnp.arange(bs, dtype=jnp.float32) % s)
    ang = t[:, None] * th_lane[None, :]
    lane = jnp.arange(_PN)
    cc = jnp.cos(ang)
    ss = jnp.sin(ang) * jnp.where(lane % 2 == 1, 1.0, -1.0)[None, :]

    proj = pl.pallas_call(
        _proj_kernel,
        grid=(nm, nn),
        in_specs=[
            pl.BlockSpec((_PM, D_MODEL), lambda i, j: (i, 0)),
            pl.BlockSpec((_PN, D_MODEL), lambda i, j: (j, 0)),
            pl.BlockSpec((_PM, _PN), lambda i, j: (i, 0)),
            pl.BlockSpec((_PM, _PN), lambda i, j: (i, 0)),
        ],
        out_specs=pl.BlockSpec((_PM, _PN), lambda i, j: (i, j)),
        out_shape=jax.ShapeDtypeStruct((bs, n_pad), jnp.float32),
    )(x_flat, W_cat, cc, ss)

    proj3 = proj.reshape(b, s, n_pad)

    qi_cb = (3 * D_MODEL) // 256
    kw_cb = (3 * D_MODEL + H_I * D_I) // 128

    groups = []
    for qq in range(nq):
        w = min(s, max(TOP_K, (qq + 1) * _MT))
        if groups and groups[-1][2] == w:
            q0, gn, _ = groups[-1]
            groups[-1] = (q0, gn + 1, w)
        else:
            groups.append((qq, 1, w))

    attn_parts = []
    for q0, gn, w in groups:
        attn_g = pl.pallas_call(
            functools.partial(_fused_kernel, q0, w, s),
            grid=(b, gn),
            in_specs=[
                pl.BlockSpec((1, _MT, D_MODEL),
                             lambda bb, j, q0=q0: (bb, q0 + j, 0)),
                pl.BlockSpec((1, _MT, H_I * D_I),
                             lambda bb, j, q0=q0: (bb, q0 + j, qi_cb)),
                pl.BlockSpec((1, _MT, 128),
                             lambda bb, j, q0=q0: (bb, q0 + j, kw_cb)),
                pl.BlockSpec((1, w, D_MODEL), lambda bb, j: (bb, 0, 1)),
                pl.BlockSpec((1, w, D_MODEL), lambda bb, j: (bb, 0, 2)),
                pl.BlockSpec((1, w, 128), lambda bb, j: (bb, 0, kw_cb)),
            ],
            out_specs=pl.BlockSpec((1, _MT, D_MODEL),
                                   lambda bb, j: (bb, j, 0)),
            out_shape=jax.ShapeDtypeStruct((b, gn * _MT, D_MODEL), jnp.float32),
        )(proj3, proj3, proj3, proj3, proj3, proj3)
        attn_parts.append(attn_g)

    attn = jnp.concatenate(attn_parts, axis=1).reshape(bs, D_MODEL)

    out = pl.pallas_call(
        _matmul_kernel,
        grid=(nm, D_MODEL // _PN),
        in_specs=[
            pl.BlockSpec((_PM, D_MODEL), lambda i, j: (i, 0)),
            pl.BlockSpec((_PN, D_MODEL), lambda i, j: (j, 0)),
        ],
        out_specs=pl.BlockSpec((_PM, _PN), lambda i, j: (i, j)),
        out_shape=jax.ShapeDtypeStruct((bs, D_MODEL), jnp.float32),
    )(attn, W_o)

    return out.reshape(b, s, D_MODEL)
